# baseline (speedup 1.0000x reference)
import functools

import jax
import jax.numpy as jnp
from jax import lax
from jax.experimental import pallas as pl
from jax.experimental.pallas import tpu as pltpu
from jax.experimental.pallas import tpu_sc as plsc

N = 10000
E = 320000
D = 128
H = 128
NW = 32
CH = 80
NT = 10240
NCH = E // (NW * CH)
EPW = NCH * CH
RPT = NT // 16
NZC = RPT // CH
SPW = (NCH + 2) * CH


def _pack_i32(v):
    return lax.bitcast_convert_type(
        v.reshape(v.shape[0], H // 2, 2), jnp.int32)


def _stage1_body(x_ref, w1a_ref, w1b_ref, b1_ref, a_ref, b_ref):
    xb = x_ref[...]
    a_ref[...] = jnp.dot(
        xb, w1a_ref[...], preferred_element_type=jnp.float32
    ).astype(jnp.bfloat16)
    b_ref[...] = (
        jnp.dot(xb, w1b_ref[...], preferred_element_type=jnp.float32)
        + b1_ref[...]
    ).astype(jnp.bfloat16)


def _stage1(x_pad, w1a, w1b, b1):
    blk = 1280
    grid = NT // blk
    return pl.pallas_call(
        _stage1_body,
        grid=(grid,),
        in_specs=[
            pl.BlockSpec((blk, D), lambda i: (i, 0)),
            pl.BlockSpec((D, H), lambda i: (0, 0)),
            pl.BlockSpec((D, H), lambda i: (0, 0)),
            pl.BlockSpec((1, H), lambda i: (0, 0)),
        ],
        out_specs=[
            pl.BlockSpec((blk, H), lambda i: (i, 0)),
            pl.BlockSpec((blk, H), lambda i: (i, 0)),
        ],
        out_shape=[
            jax.ShapeDtypeStruct((NT, H), jnp.bfloat16),
            jax.ShapeDtypeStruct((NT, H), jnp.bfloat16),
        ],
    )(x_pad, w1a, w1b, b1)


def _sc_body(a_hbm, b_hbm, x_hbm, rowi_hbm, coli_hbm, w2_hbm, b2_hbm,
             out_hbm, acc, ridx0, ridx1, cidx0, cidx1, a_v0, a_v1, b_v0,
             b_v1, x_v0, x_v1, w2_v, b2_v, t_v, wch_v, semd0, semd1,
             semi0, semi1):
    cid = lax.axis_index("c")
    sid = lax.axis_index("s")
    wid = cid * 16 + sid

    a_s = (a_v0, a_v1)
    b_s = (b_v0, b_v1)
    x_s = (x_v0, x_v1)
    ridx = (ridx0, ridx1)
    cidx = (cidx0, cidx1)
    semd = (semd0, semd1)
    semi = (semi0, semi1)

    pltpu.sync_copy(w2_hbm, w2_v)
    pltpu.sync_copy(b2_hbm, b2_v)

    def _zrow(r, c):
        for j in range(8):
            x_v0[r, pl.ds(16 * j, 16)] = jnp.zeros((16,), jnp.float32)
        return c

    lax.fori_loop(0, CH, _zrow, 0)
    for t in range(NZC):
        pltpu.sync_copy(x_v0, acc.at[pl.ds(sid * RPT + t * CH, CH)])
    plsc.subcore_barrier()

    def _issue_idx(c, slot):
        base = wid * SPW + c * CH
        pltpu.async_copy(rowi_hbm.at[pl.ds(base, CH)], ridx[slot],
                         semi[slot])
        pltpu.async_copy(coli_hbm.at[pl.ds(base, CH)], cidx[slot],
                         semi[slot])

    def _wait_idx(c, slot):
        base = wid * SPW + c * CH
        pltpu.make_async_copy(rowi_hbm.at[pl.ds(base, CH)], ridx[slot],
                              semi[slot]).wait()
        pltpu.make_async_copy(coli_hbm.at[pl.ds(base, CH)], cidx[slot],
                              semi[slot]).wait()

    def _issue(slot):
        pltpu.async_copy(a_hbm.at[ridx[slot]], a_s[slot], semd[slot])
        pltpu.async_copy(b_hbm.at[cidx[slot]], b_s[slot], semd[slot])
        pltpu.async_copy(x_hbm.at[cidx[slot]], x_s[slot], semd[slot])

    def _wait(slot):
        pltpu.make_async_copy(
            a_hbm.at[ridx[slot]], a_s[slot], semd[slot]).wait()
        pltpu.make_async_copy(
            b_hbm.at[cidx[slot]], b_s[slot], semd[slot]).wait()
        pltpu.make_async_copy(
            x_hbm.at[cidx[slot]], x_s[slot], semd[slot]).wait()

    def _compute(slot):
        a_r, b_r, x_r = a_s[slot], b_s[slot], x_s[slot]
        w2r = [w2_v[pl.ds(32 * j, 32)] for j in range(4)]
        lane = lax.iota(jnp.int32, 16)

        zero_bf = jnp.zeros((32,), jnp.bfloat16)

        @plsc.parallel_loop(0, CH, step=1, unroll=8)
        def _p1(e):
            accv = zero_bf
            for j in range(4):
                s = pl.ds(16 * j, 16)
                aj = plsc.bitcast(a_r[e, s], jnp.bfloat16)
                bj = plsc.bitcast(b_r[e, s], jnp.bfloat16)
                accv = accv + jnp.maximum(aj + bj, zero_bf) * w2r[j]
            lo, hi = plsc.unpack(accv, format=plsc.PackFormat.INTERLEAVED)
            t_v[pl.ds(e * 16, 16)] = lo + hi

        b2r = b2_v[...]
        lane16 = lane * 16
        for g in range(CH // 16):
            gs = [plsc.load_gather(t_v, [lane16 + (256 * g + k)])
                  for k in range(16)]
            while len(gs) > 1:
                gs = [gs[i] + gs[i + 1] for i in range(0, len(gs), 2)]
            wg = 1.0 / (1.0 + jnp.exp(-(b2r + gs[0])))
            wch_v[pl.ds(16 * g, 16)] = wg

        @plsc.parallel_loop(0, CH, step=1, unroll=8)
        def _p3(e):
            wb = plsc.load_gather(wch_v, [jnp.broadcast_to(e, (16,))])
            for j in range(8):
                s = pl.ds(16 * j, 16)
                x_r[e, s] = wb * x_r[e, s]

    def _step(c, sl):
        _wait_idx(c + 1, 1 - sl)
        _issue(1 - sl)
        _wait(sl)
        _compute(sl)
        pltpu.sync_copy(x_s[sl], acc.at[ridx[sl]], add=True)
        _issue_idx(c + 2, sl)

    _issue_idx(0, 0)
    _issue_idx(1, 1)
    _wait_idx(0, 0)
    _issue(0)

    def _pair(i, carry):
        for sl in (0, 1):
            _step(2 * i + sl, sl)
        return carry

    lax.fori_loop(0, NCH // 2, _pair, 0)
    if NCH % 2:
        _step(NCH - 1, 0)
    _wait(NCH % 2)
    _wait_idx(NCH + 1, 1 - (NCH % 2))
    plsc.subcore_barrier()

    for t in range(NZC):
        r0 = sid * RPT + t * CH
        pltpu.sync_copy(acc.at[pl.ds(r0, CH)], x_v0)
        pltpu.sync_copy(x_v0, out_hbm.at[cid, pl.ds(r0, CH)])


_sc_kernel = pl.kernel(
    _sc_body,
    out_type=jax.ShapeDtypeStruct((2, NT, D), jnp.float32),
    mesh=plsc.VectorSubcoreMesh(core_axis_name="c", subcore_axis_name="s"),
    compiler_params=pltpu.CompilerParams(
        needs_layout_passes=False, use_tc_tiling_on_sc=False),
    scratch_types=[
        pltpu.VMEM_SHARED((NT, D), jnp.float32),
        pltpu.VMEM((CH,), jnp.int32),
        pltpu.VMEM((CH,), jnp.int32),
        pltpu.VMEM((CH,), jnp.int32),
        pltpu.VMEM((CH,), jnp.int32),
        pltpu.VMEM((CH, H // 2), jnp.int32),
        pltpu.VMEM((CH, H // 2), jnp.int32),
        pltpu.VMEM((CH, H // 2), jnp.int32),
        pltpu.VMEM((CH, H // 2), jnp.int32),
        pltpu.VMEM((CH, D), jnp.float32),
        pltpu.VMEM((CH, D), jnp.float32),
        pltpu.VMEM((H,), jnp.bfloat16),
        pltpu.VMEM((16,), jnp.float32),
        pltpu.VMEM((CH * 16,), jnp.float32),
        pltpu.VMEM((CH,), jnp.float32),
        pltpu.SemaphoreType.DMA,
        pltpu.SemaphoreType.DMA,
        pltpu.SemaphoreType.DMA,
        pltpu.SemaphoreType.DMA,
    ],
)


def _stage3_body(p0_ref, p1_ref, o_ref):
    o_ref[...] = p0_ref[...] + p1_ref[...]


def _stage3(p0, p1):
    blk = 1000
    grid = N // blk
    return pl.pallas_call(
        _stage3_body,
        grid=(grid,),
        in_specs=[
            pl.BlockSpec((blk, D), lambda i: (i, 0)),
            pl.BlockSpec((blk, D), lambda i: (i, 0)),
        ],
        out_specs=pl.BlockSpec((blk, D), lambda i: (i, 0)),
        out_shape=jax.ShapeDtypeStruct((N, D), jnp.float32),
    )(p0, p1)


@jax.jit
def kernel(x, edge_index, W1, b1, W2, b2):
    x_pad = jnp.zeros((NT, D), jnp.float32).at[:N].set(x)
    w1a = W1[:D]
    w1b = W1[D:]
    a, b = _stage1(x_pad, w1a, w1b, b1.reshape(1, H))
    a = _pack_i32(a)
    b = _pack_i32(b)

    ei = edge_index.astype(jnp.int32)
    zpad = jnp.zeros((NW, 2 * CH), jnp.int32)
    rowi = jnp.concatenate([ei[0].reshape(NW, EPW), zpad], axis=1).reshape(-1)
    coli = jnp.concatenate([ei[1].reshape(NW, EPW), zpad], axis=1).reshape(-1)

    w2f = W2.reshape(H).astype(jnp.bfloat16)
    b2v = jnp.full((16,), b2[0], jnp.float32)

    parts = _sc_kernel(a, b, x_pad, rowi, coli, w2f, b2v)
    return _stage3(parts[0, :N], parts[1, :N])


# speedup vs baseline: 1.0049x; 1.0049x over previous
import functools

import jax
import jax.numpy as jnp
from jax import lax
from jax.experimental import pallas as pl
from jax.experimental.pallas import tpu as pltpu
from jax.experimental.pallas import tpu_sc as plsc

N = 10000
E = 320000
D = 128
H = 128
NW = 32
CH = 80
NT = 10240
NCH = E // (NW * CH)
EPW = NCH * CH
RPT = NT // 16
NZC = RPT // CH
SPW = (NCH + 2) * CH


def _pack_i32(v):
    return lax.bitcast_convert_type(
        v.reshape(v.shape[0], H // 2, 2), jnp.int32)


def _stage1_body(x_ref, w1a_ref, w1b_ref, b1_ref, a_ref, b_ref):
    xb = x_ref[...]
    a_ref[...] = jnp.dot(
        xb, w1a_ref[...], preferred_element_type=jnp.float32
    ).astype(jnp.bfloat16)
    b_ref[...] = (
        jnp.dot(xb, w1b_ref[...], preferred_element_type=jnp.float32)
        + b1_ref[...]
    ).astype(jnp.bfloat16)


def _stage1(x_pad, w1a, w1b, b1):
    blk = 1280
    grid = NT // blk
    return pl.pallas_call(
        _stage1_body,
        grid=(grid,),
        in_specs=[
            pl.BlockSpec((blk, D), lambda i: (i, 0)),
            pl.BlockSpec((D, H), lambda i: (0, 0)),
            pl.BlockSpec((D, H), lambda i: (0, 0)),
            pl.BlockSpec((1, H), lambda i: (0, 0)),
        ],
        out_specs=[
            pl.BlockSpec((blk, H), lambda i: (i, 0)),
            pl.BlockSpec((blk, H), lambda i: (i, 0)),
        ],
        out_shape=[
            jax.ShapeDtypeStruct((NT, H), jnp.bfloat16),
            jax.ShapeDtypeStruct((NT, H), jnp.bfloat16),
        ],
    )(x_pad, w1a, w1b, b1)


def _sc_body(a_hbm, b_hbm, x_hbm, rowi_hbm, coli_hbm, w2_hbm, b2_hbm,
             out_hbm, acc, ridx0, ridx1, cidx0, cidx1, a_v0, a_v1, b_v0,
             b_v1, x_v0, x_v1, w2_v, b2_v, t_v, wch_v, semd0, semd1,
             semi0, semi1):
    cid = lax.axis_index("c")
    sid = lax.axis_index("s")
    wid = cid * 16 + sid

    a_s = (a_v0, a_v1)
    b_s = (b_v0, b_v1)
    x_s = (x_v0, x_v1)
    ridx = (ridx0, ridx1)
    cidx = (cidx0, cidx1)
    semd = (semd0, semd1)
    semi = (semi0, semi1)

    pltpu.sync_copy(w2_hbm, w2_v)
    pltpu.sync_copy(b2_hbm, b2_v)

    def _zrow(r, c):
        for j in range(8):
            x_v0[r, pl.ds(16 * j, 16)] = jnp.zeros((16,), jnp.float32)
        return c

    lax.fori_loop(0, CH, _zrow, 0)
    for t in range(NZC):
        pltpu.sync_copy(x_v0, acc.at[pl.ds(sid * RPT + t * CH, CH)])
    plsc.subcore_barrier()

    def _issue_idx(c, slot):
        base = wid * SPW + c * CH
        pltpu.async_copy(rowi_hbm.at[pl.ds(base, CH)], ridx[slot],
                         semi[slot])
        pltpu.async_copy(coli_hbm.at[pl.ds(base, CH)], cidx[slot],
                         semi[slot])

    def _wait_idx(c, slot):
        base = wid * SPW + c * CH
        pltpu.make_async_copy(rowi_hbm.at[pl.ds(base, CH)], ridx[slot],
                              semi[slot]).wait()
        pltpu.make_async_copy(coli_hbm.at[pl.ds(base, CH)], cidx[slot],
                              semi[slot]).wait()

    def _issue(slot):
        pltpu.async_copy(a_hbm.at[ridx[slot]], a_s[slot], semd[slot])
        pltpu.async_copy(b_hbm.at[cidx[slot]], b_s[slot], semd[slot])
        pltpu.async_copy(x_hbm.at[cidx[slot]], x_s[slot], semd[slot])

    def _wait(slot):
        pltpu.make_async_copy(
            a_hbm.at[ridx[slot]], a_s[slot], semd[slot]).wait()
        pltpu.make_async_copy(
            b_hbm.at[cidx[slot]], b_s[slot], semd[slot]).wait()
        pltpu.make_async_copy(
            x_hbm.at[cidx[slot]], x_s[slot], semd[slot]).wait()

    def _compute(slot):
        a_r, b_r, x_r = a_s[slot], b_s[slot], x_s[slot]
        w2r = [w2_v[pl.ds(32 * j, 32)] for j in range(4)]
        lane = lax.iota(jnp.int32, 16)

        zero_bf = jnp.zeros((32,), jnp.bfloat16)

        @plsc.parallel_loop(0, CH, step=1, unroll=4)
        def _p1(e):
            ps = []
            for j in range(4):
                s = pl.ds(16 * j, 16)
                aj = plsc.bitcast(a_r[e, s], jnp.bfloat16)
                bj = plsc.bitcast(b_r[e, s], jnp.bfloat16)
                ps.append(jnp.maximum(aj + bj, zero_bf) * w2r[j])
            accv = (ps[0] + ps[1]) + (ps[2] + ps[3])
            lo, hi = plsc.unpack(accv, format=plsc.PackFormat.INTERLEAVED)
            t_v[pl.ds(e * 16, 16)] = lo + hi

        b2r = b2_v[...]
        lane16 = lane * 16
        for g in range(CH // 16):
            gs = [plsc.load_gather(t_v, [lane16 + (256 * g + k)])
                  for k in range(16)]
            while len(gs) > 1:
                gs = [gs[i] + gs[i + 1] for i in range(0, len(gs), 2)]
            wg = 1.0 / (1.0 + jnp.exp(-(b2r + gs[0])))
            wch_v[pl.ds(16 * g, 16)] = wg

        @plsc.parallel_loop(0, CH, step=1, unroll=4)
        def _p3(e):
            wb = plsc.load_gather(wch_v, [jnp.broadcast_to(e, (16,))])
            for j in range(8):
                s = pl.ds(16 * j, 16)
                x_r[e, s] = wb * x_r[e, s]

    def _step(c, sl):
        _wait_idx(c + 1, 1 - sl)
        _issue(1 - sl)
        _wait(sl)
        _compute(sl)
        pltpu.sync_copy(x_s[sl], acc.at[ridx[sl]], add=True)
        _issue_idx(c + 2, sl)

    _issue_idx(0, 0)
    _issue_idx(1, 1)
    _wait_idx(0, 0)
    _issue(0)

    def _pair(i, carry):
        for sl in (0, 1):
            _step(2 * i + sl, sl)
        return carry

    lax.fori_loop(0, NCH // 2, _pair, 0)
    if NCH % 2:
        _step(NCH - 1, 0)
    _wait(NCH % 2)
    _wait_idx(NCH + 1, 1 - (NCH % 2))
    plsc.subcore_barrier()

    for t in range(NZC):
        r0 = sid * RPT + t * CH
        pltpu.sync_copy(acc.at[pl.ds(r0, CH)], x_v0)
        pltpu.sync_copy(x_v0, out_hbm.at[cid, pl.ds(r0, CH)])


_sc_kernel = pl.kernel(
    _sc_body,
    out_type=jax.ShapeDtypeStruct((2, NT, D), jnp.float32),
    mesh=plsc.VectorSubcoreMesh(core_axis_name="c", subcore_axis_name="s"),
    compiler_params=pltpu.CompilerParams(
        needs_layout_passes=False, use_tc_tiling_on_sc=False),
    scratch_types=[
        pltpu.VMEM_SHARED((NT, D), jnp.float32),
        pltpu.VMEM((CH,), jnp.int32),
        pltpu.VMEM((CH,), jnp.int32),
        pltpu.VMEM((CH,), jnp.int32),
        pltpu.VMEM((CH,), jnp.int32),
        pltpu.VMEM((CH, H // 2), jnp.int32),
        pltpu.VMEM((CH, H // 2), jnp.int32),
        pltpu.VMEM((CH, H // 2), jnp.int32),
        pltpu.VMEM((CH, H // 2), jnp.int32),
        pltpu.VMEM((CH, D), jnp.float32),
        pltpu.VMEM((CH, D), jnp.float32),
        pltpu.VMEM((H,), jnp.bfloat16),
        pltpu.VMEM((16,), jnp.float32),
        pltpu.VMEM((CH * 16,), jnp.float32),
        pltpu.VMEM((CH,), jnp.float32),
        pltpu.SemaphoreType.DMA,
        pltpu.SemaphoreType.DMA,
        pltpu.SemaphoreType.DMA,
        pltpu.SemaphoreType.DMA,
    ],
)


def _stage3_body(p0_ref, p1_ref, o_ref):
    o_ref[...] = p0_ref[...] + p1_ref[...]


def _stage3(p0, p1):
    blk = 1000
    grid = N // blk
    return pl.pallas_call(
        _stage3_body,
        grid=(grid,),
        in_specs=[
            pl.BlockSpec((blk, D), lambda i: (i, 0)),
            pl.BlockSpec((blk, D), lambda i: (i, 0)),
        ],
        out_specs=pl.BlockSpec((blk, D), lambda i: (i, 0)),
        out_shape=jax.ShapeDtypeStruct((N, D), jnp.float32),
    )(p0, p1)


@jax.jit
def kernel(x, edge_index, W1, b1, W2, b2):
    x_pad = jnp.zeros((NT, D), jnp.float32).at[:N].set(x)
    w1a = W1[:D]
    w1b = W1[D:]
    a, b = _stage1(x_pad, w1a, w1b, b1.reshape(1, H))
    a = _pack_i32(a)
    b = _pack_i32(b)

    ei = edge_index.astype(jnp.int32)
    zpad = jnp.zeros((NW, 2 * CH), jnp.int32)
    rowi = jnp.concatenate([ei[0].reshape(NW, EPW), zpad], axis=1).reshape(-1)
    coli = jnp.concatenate([ei[1].reshape(NW, EPW), zpad], axis=1).reshape(-1)

    w2f = W2.reshape(H).astype(jnp.bfloat16)
    b2v = jnp.full((16,), b2[0], jnp.float32)

    parts = _sc_kernel(a, b, x_pad, rowi, coli, w2f, b2v)
    return _stage3(parts[0, :N], parts[1, :N])
